# Initial kernel scaffold; baseline (speedup 1.0000x reference)
import jax
import jax.numpy as jnp
from jax.experimental import pallas as pl


def kernel(r, offsets, idx_i, idx_j):
    raise NotImplementedError("write your pallas kernel here")


# speedup vs baseline: 5.7949x; 5.7949x over previous
import functools

import jax
import jax.numpy as jnp
from jax import lax
from jax.experimental import pallas as pl
from jax.experimental.pallas import tpu as pltpu
from jax.experimental.pallas import tpu_sc as plsc

NC = 2
NS = 16
LANES = 16


def _dist_body(n_nodes, n_edges, chunk,
               rx_hbm, ry_hbm, rz_hbm, off_hbm, ii_hbm, jj_hbm, out_hbm,
               rx_s, ry_s, rz_s,
               ii_v, jj_v, off_v, gxi, gyi, gzi, gxj, gyj, gzj, out_v,
               sem_g):
    cid = lax.axis_index("c")
    sid = lax.axis_index("s")
    wid = cid * NS + sid

    @pl.when(sid == 0)
    def _stage():
        pltpu.sync_copy(rx_hbm, rx_s)
        pltpu.sync_copy(ry_hbm, ry_s)
        pltpu.sync_copy(rz_hbm, rz_s)

    plsc.subcore_barrier()

    ept = n_edges // (NC * NS)
    nch = ept // chunk
    groups = chunk // LANES

    iota = lax.iota(jnp.int32, LANES)
    io3 = iota * 3

    def chunk_body(ci, _):
        base = wid * ept + ci * chunk
        pltpu.sync_copy(ii_hbm.at[pl.ds(base, chunk)], ii_v)
        pltpu.sync_copy(jj_hbm.at[pl.ds(base, chunk)], jj_v)
        pltpu.sync_copy(off_hbm.at[pl.ds(base * 3, chunk * 3)], off_v)

        cds = [
            pltpu.async_copy(rx_s.at[ii_v], gxi, sem_g),
            pltpu.async_copy(ry_s.at[ii_v], gyi, sem_g),
            pltpu.async_copy(rz_s.at[ii_v], gzi, sem_g),
            pltpu.async_copy(rx_s.at[jj_v], gxj, sem_g),
            pltpu.async_copy(ry_s.at[jj_v], gyj, sem_g),
            pltpu.async_copy(rz_s.at[jj_v], gzj, sem_g),
        ]
        for cd in cds:
            cd.wait()

        def group_body(g, _):
            lin = g * LANES
            xi = gxi[pl.ds(lin, LANES)]
            yi = gyi[pl.ds(lin, LANES)]
            zi = gzi[pl.ds(lin, LANES)]
            xj = gxj[pl.ds(lin, LANES)]
            yj = gyj[pl.ds(lin, LANES)]
            zj = gzj[pl.ds(lin, LANES)]
            b3 = g * (3 * LANES)
            ox = plsc.load_gather(off_v, [io3 + b3])
            oy = plsc.load_gather(off_v, [io3 + (b3 + 1)])
            oz = plsc.load_gather(off_v, [io3 + (b3 + 2)])
            dx = xi - (xj + ox)
            dy = yi - (yj + oy)
            dz = zi - (zj + oz)
            s = dx * dx + dy * dy + dz * dz
            s = jnp.maximum(s, jnp.float32(1e-30))
            i = plsc.bitcast(s, jnp.int32)
            i = jnp.int32(0x5F3759DF) - (i >> 1)
            y = plsc.bitcast(i, jnp.float32)
            hs = s * jnp.float32(0.5)
            y = y * (jnp.float32(1.5) - hs * y * y)
            y = y * (jnp.float32(1.5) - hs * y * y)
            y = y * (jnp.float32(1.5) - hs * y * y)
            out_v[pl.ds(lin, LANES)] = s * y
            return ()

        lax.fori_loop(0, groups, group_body, (), unroll=2)
        pltpu.sync_copy(out_v, out_hbm.at[pl.ds(base, chunk)])
        return ()

    lax.fori_loop(0, nch, chunk_body, ())


@functools.partial(jax.jit, static_argnames=("n_nodes", "n_edges", "chunk"))
def _dist(rx, ry, rz, off_flat, ii, jj, *, n_nodes, n_edges, chunk):
    mesh = plsc.VectorSubcoreMesh(
        core_axis_name="c", subcore_axis_name="s",
        num_cores=NC, num_subcores=NS)
    body = functools.partial(_dist_body, n_nodes, n_edges, chunk)
    return pl.kernel(
        body,
        out_type=jax.ShapeDtypeStruct((n_edges,), jnp.float32),
        mesh=mesh,
        compiler_params=pltpu.CompilerParams(needs_layout_passes=False),
        scratch_types=[
            pltpu.VMEM_SHARED((n_nodes,), jnp.float32),
            pltpu.VMEM_SHARED((n_nodes,), jnp.float32),
            pltpu.VMEM_SHARED((n_nodes,), jnp.float32),
            pltpu.VMEM((chunk,), jnp.int32),
            pltpu.VMEM((chunk,), jnp.int32),
            pltpu.VMEM((chunk * 3,), jnp.float32),
            pltpu.VMEM((chunk,), jnp.float32),
            pltpu.VMEM((chunk,), jnp.float32),
            pltpu.VMEM((chunk,), jnp.float32),
            pltpu.VMEM((chunk,), jnp.float32),
            pltpu.VMEM((chunk,), jnp.float32),
            pltpu.VMEM((chunk,), jnp.float32),
            pltpu.VMEM((chunk,), jnp.float32),
            pltpu.SemaphoreType.DMA,
        ],
    )(rx, ry, rz, off_flat, ii, jj)


def kernel(r, offsets, idx_i, idx_j):
    r = r.astype(jnp.float32)
    n_nodes = r.shape[0]
    n_edges = idx_i.shape[0]
    rx = r[:, 0]
    ry = r[:, 1]
    rz = r[:, 2]
    off_flat = offsets.astype(jnp.float32).reshape(-1)
    ii = idx_i.astype(jnp.int32)
    jj = idx_j.astype(jnp.int32)
    out = _dist(rx, ry, rz, off_flat, ii, jj,
                n_nodes=n_nodes, n_edges=n_edges, chunk=2000)
    return out.reshape(n_edges, 1)


# speedup vs baseline: 45.4042x; 7.8352x over previous
import functools

import jax
import jax.numpy as jnp
from jax import lax
from jax.experimental import pallas as pl
from jax.experimental.pallas import tpu as pltpu
from jax.experimental.pallas import tpu_sc as plsc

NC = 2
NS = 16
LANES = 16


def _dist_body(n_nodes, n_edges, chunk,
               rx_hbm, ry_hbm, rz_hbm, ox_hbm, oy_hbm, oz_hbm,
               ii_hbm, jj_hbm, out_hbm,
               rx_s, ry_s, rz_s,
               ii_v, jj_v, ox_v, oy_v, oz_v,
               gxi, gyi, gzi, gxj, gyj, gzj, out_v,
               sem_g):
    cid = lax.axis_index("c")
    sid = lax.axis_index("s")
    wid = cid * NS + sid

    @pl.when(sid == 0)
    def _stage():
        pltpu.sync_copy(rx_hbm, rx_s)
        pltpu.sync_copy(ry_hbm, ry_s)
        pltpu.sync_copy(rz_hbm, rz_s)

    plsc.subcore_barrier()

    ept = n_edges // (NC * NS)
    nch = ept // chunk
    groups = chunk // LANES

    def chunk_body(ci, _):
        base = wid * ept + ci * chunk
        pltpu.sync_copy(ii_hbm.at[pl.ds(base, chunk)], ii_v)
        pltpu.sync_copy(jj_hbm.at[pl.ds(base, chunk)], jj_v)
        pltpu.sync_copy(ox_hbm.at[pl.ds(base, chunk)], ox_v)
        pltpu.sync_copy(oy_hbm.at[pl.ds(base, chunk)], oy_v)
        pltpu.sync_copy(oz_hbm.at[pl.ds(base, chunk)], oz_v)

        cds = [
            pltpu.async_copy(rx_s.at[ii_v], gxi, sem_g),
            pltpu.async_copy(ry_s.at[ii_v], gyi, sem_g),
            pltpu.async_copy(rz_s.at[ii_v], gzi, sem_g),
            pltpu.async_copy(rx_s.at[jj_v], gxj, sem_g),
            pltpu.async_copy(ry_s.at[jj_v], gyj, sem_g),
            pltpu.async_copy(rz_s.at[jj_v], gzj, sem_g),
        ]
        for cd in cds:
            cd.wait()

        def group_body(g, _):
            lin = g * LANES
            xi = gxi[pl.ds(lin, LANES)]
            yi = gyi[pl.ds(lin, LANES)]
            zi = gzi[pl.ds(lin, LANES)]
            xj = gxj[pl.ds(lin, LANES)]
            yj = gyj[pl.ds(lin, LANES)]
            zj = gzj[pl.ds(lin, LANES)]
            ox = ox_v[pl.ds(lin, LANES)]
            oy = oy_v[pl.ds(lin, LANES)]
            oz = oz_v[pl.ds(lin, LANES)]
            dx = xi - (xj + ox)
            dy = yi - (yj + oy)
            dz = zi - (zj + oz)
            s = dx * dx + dy * dy + dz * dz
            s = jnp.maximum(s, jnp.float32(1e-30))
            i = plsc.bitcast(s, jnp.int32)
            i = jnp.int32(0x5F3759DF) - (i >> 1)
            y = plsc.bitcast(i, jnp.float32)
            hs = s * jnp.float32(0.5)
            y = y * (jnp.float32(1.5) - hs * y * y)
            y = y * (jnp.float32(1.5) - hs * y * y)
            y = y * (jnp.float32(1.5) - hs * y * y)
            out_v[pl.ds(lin, LANES)] = s * y
            return ()

        lax.fori_loop(0, groups, group_body, (), unroll=2)
        pltpu.sync_copy(out_v, out_hbm.at[pl.ds(base, chunk)])
        return ()

    lax.fori_loop(0, nch, chunk_body, ())


@functools.partial(jax.jit, static_argnames=("n_nodes", "n_edges", "chunk"))
def _dist(rx, ry, rz, ox, oy, oz, ii, jj, *, n_nodes, n_edges, chunk):
    mesh = plsc.VectorSubcoreMesh(
        core_axis_name="c", subcore_axis_name="s",
        num_cores=NC, num_subcores=NS)
    body = functools.partial(_dist_body, n_nodes, n_edges, chunk)
    return pl.kernel(
        body,
        out_type=jax.ShapeDtypeStruct((n_edges,), jnp.float32),
        mesh=mesh,
        compiler_params=pltpu.CompilerParams(needs_layout_passes=False),
        scratch_types=[
            pltpu.VMEM_SHARED((n_nodes,), jnp.float32),
            pltpu.VMEM_SHARED((n_nodes,), jnp.float32),
            pltpu.VMEM_SHARED((n_nodes,), jnp.float32),
            pltpu.VMEM((chunk,), jnp.int32),
            pltpu.VMEM((chunk,), jnp.int32),
            pltpu.VMEM((chunk,), jnp.float32),
            pltpu.VMEM((chunk,), jnp.float32),
            pltpu.VMEM((chunk,), jnp.float32),
            pltpu.VMEM((chunk,), jnp.float32),
            pltpu.VMEM((chunk,), jnp.float32),
            pltpu.VMEM((chunk,), jnp.float32),
            pltpu.VMEM((chunk,), jnp.float32),
            pltpu.VMEM((chunk,), jnp.float32),
            pltpu.VMEM((chunk,), jnp.float32),
            pltpu.VMEM((chunk,), jnp.float32),
            pltpu.SemaphoreType.DMA,
        ],
    )(rx, ry, rz, ox, oy, oz, ii, jj)


def kernel(r, offsets, idx_i, idx_j):
    r = r.astype(jnp.float32)
    offsets = offsets.astype(jnp.float32)
    n_nodes = r.shape[0]
    n_edges = idx_i.shape[0]
    rx, ry, rz = r[:, 0], r[:, 1], r[:, 2]
    ox, oy, oz = offsets[:, 0], offsets[:, 1], offsets[:, 2]
    ii = idx_i.astype(jnp.int32)
    jj = idx_j.astype(jnp.int32)
    out = _dist(rx, ry, rz, ox, oy, oz, ii, jj,
                n_nodes=n_nodes, n_edges=n_edges, chunk=2000)
    return out.reshape(n_edges, 1)


# speedup vs baseline: 100.8127x; 2.2203x over previous
import functools

import jax
import jax.numpy as jnp
from jax import lax
from jax.experimental import pallas as pl
from jax.experimental.pallas import tpu as pltpu
from jax.experimental.pallas import tpu_sc as plsc

NC = 2
NS = 16
LANES = 16
NB = 5


def _dist_body(n_nodes, n_edges, chunk,
               rx_hbm, ry_hbm, rz_hbm, ox_hbm, oy_hbm, oz_hbm,
               ii_hbm, jj_hbm, out_hbm,
               rx_s, ry_s, rz_s,
               ii_v, jj_v, ox_v, oy_v, oz_v,
               gxi, gyi, gzi, gxj, gyj, gzj, out_v,
               sem_load, sem_gath, sem_out):
    cid = lax.axis_index("c")
    sid = lax.axis_index("s")
    wid = cid * NS + sid

    @pl.when(sid == 0)
    def _stage():
        pltpu.sync_copy(rx_hbm, rx_s)
        pltpu.sync_copy(ry_hbm, ry_s)
        pltpu.sync_copy(rz_hbm, rz_s)

    plsc.subcore_barrier()

    ept = n_edges // (NC * NS)
    nch = ept // chunk
    groups = chunk // LANES
    assert chunk % LANES == 0 and ept % chunk == 0 and nch % NB == 0
    tile0 = wid * ept

    lin_pairs = lambda b: (
        (ii_hbm, ii_v[b]), (jj_hbm, jj_v[b]),
        (ox_hbm, ox_v[b]), (oy_hbm, oy_v[b]), (oz_hbm, oz_v[b]))

    def fire_loads(ci, b):
        base = tile0 + ci * chunk
        for src, dst in lin_pairs(b):
            pltpu.async_copy(src.at[pl.ds(base, chunk)], dst, sem_load.at[b])

    def wait_loads(b):
        for src, dst in lin_pairs(b):
            pltpu.make_async_copy(src.at[pl.ds(0, chunk)], dst, sem_load.at[b]).wait()

    def gath_triples(b):
        return ((rx_s, ii_v[b], gxi[b]), (ry_s, ii_v[b], gyi[b]),
                (rz_s, ii_v[b], gzi[b]), (rx_s, jj_v[b], gxj[b]),
                (ry_s, jj_v[b], gyj[b]), (rz_s, jj_v[b], gzj[b]))

    def fire_gathers(b):
        for tab, idx, dst in gath_triples(b):
            pltpu.async_copy(tab.at[idx], dst, sem_gath.at[b])

    def wait_gathers(b):
        for tab, idx, dst in gath_triples(b):
            pltpu.make_async_copy(tab.at[idx], dst, sem_gath.at[b]).wait()

    def fire_store(ci, b):
        base = tile0 + ci * chunk
        pltpu.async_copy(out_v[b], out_hbm.at[pl.ds(base, chunk)], sem_out.at[b])

    def wait_store(b):
        pltpu.make_async_copy(out_v[b], out_hbm.at[pl.ds(0, chunk)], sem_out.at[b]).wait()

    def compute(b):
        def group_body(g, _):
            lin = g * LANES
            xi = gxi[b][pl.ds(lin, LANES)]
            yi = gyi[b][pl.ds(lin, LANES)]
            zi = gzi[b][pl.ds(lin, LANES)]
            xj = gxj[b][pl.ds(lin, LANES)]
            yj = gyj[b][pl.ds(lin, LANES)]
            zj = gzj[b][pl.ds(lin, LANES)]
            ox = ox_v[b][pl.ds(lin, LANES)]
            oy = oy_v[b][pl.ds(lin, LANES)]
            oz = oz_v[b][pl.ds(lin, LANES)]
            dx = xi - (xj + ox)
            dy = yi - (yj + oy)
            dz = zi - (zj + oz)
            s = dx * dx + dy * dy + dz * dz
            s = jnp.maximum(s, jnp.float32(1e-30))
            i = plsc.bitcast(s, jnp.int32)
            i = jnp.int32(0x5F3759DF) - (i >> 1)
            y = plsc.bitcast(i, jnp.float32)
            hs = s * jnp.float32(0.5)
            y = y * (jnp.float32(1.5) - hs * y * y)
            y = y * (jnp.float32(1.5) - hs * y * y)
            y = y * (jnp.float32(1.5) - hs * y * y)
            out_v[b][pl.ds(lin, LANES)] = s * y
            return ()

        lax.fori_loop(0, groups, group_body, (), unroll=4)

    for b in range(NB - 1):
        fire_loads(b, b)
    wait_loads(0)
    fire_gathers(0)

    def step(s, _):
        for b in range(NB):
            ci = s * NB + b

            wait_gathers(b)

            bn = (b + 1) % NB

            @pl.when(ci + 1 < nch)
            def _next_gath():
                wait_loads(bn)
                fire_gathers(bn)

            bl = (b + NB - 1) % NB

            @pl.when(ci + (NB - 1) < nch)
            def _next_loads():
                fire_loads(ci + (NB - 1), bl)

            @pl.when(ci >= NB)
            def _drain_store():
                wait_store(b)

            compute(b)
            fire_store(ci, b)
        return ()

    lax.fori_loop(0, nch // NB, step, ())

    for b in range(NB):
        wait_store(b)


@functools.partial(jax.jit, static_argnames=("n_nodes", "n_edges", "chunk"))
def _dist(rx, ry, rz, ox, oy, oz, ii, jj, *, n_nodes, n_edges, chunk):
    mesh = plsc.VectorSubcoreMesh(
        core_axis_name="c", subcore_axis_name="s",
        num_cores=NC, num_subcores=NS)
    body = functools.partial(_dist_body, n_nodes, n_edges, chunk)
    vf = lambda: [pltpu.VMEM((chunk,), jnp.float32) for _ in range(NB)]
    vi = lambda: [pltpu.VMEM((chunk,), jnp.int32) for _ in range(NB)]
    return pl.kernel(
        body,
        out_type=jax.ShapeDtypeStruct((n_edges,), jnp.float32),
        mesh=mesh,
        compiler_params=pltpu.CompilerParams(needs_layout_passes=False),
        scratch_types=[
            pltpu.VMEM_SHARED((n_nodes,), jnp.float32),
            pltpu.VMEM_SHARED((n_nodes,), jnp.float32),
            pltpu.VMEM_SHARED((n_nodes,), jnp.float32),
            vi(), vi(), vf(), vf(), vf(),
            vf(), vf(), vf(), vf(), vf(), vf(), vf(),
            pltpu.SemaphoreType.DMA((NB,)),
            pltpu.SemaphoreType.DMA((NB,)),
            pltpu.SemaphoreType.DMA((NB,)),
        ],
    )(rx, ry, rz, ox, oy, oz, ii, jj)


def kernel(r, offsets, idx_i, idx_j):
    r = r.astype(jnp.float32)
    offsets = offsets.astype(jnp.float32)
    n_nodes = r.shape[0]
    n_edges = idx_i.shape[0]
    rx, ry, rz = r[:, 0], r[:, 1], r[:, 2]
    ox, oy, oz = offsets[:, 0], offsets[:, 1], offsets[:, 2]
    ii = idx_i.astype(jnp.int32)
    jj = idx_j.astype(jnp.int32)
    out = _dist(rx, ry, rz, ox, oy, oz, ii, jj,
                n_nodes=n_nodes, n_edges=n_edges, chunk=800)
    return out.reshape(n_edges, 1)


# speedup vs baseline: 103.0576x; 1.0223x over previous
import functools

import jax
import jax.numpy as jnp
from jax import lax
from jax.experimental import pallas as pl
from jax.experimental.pallas import tpu as pltpu
from jax.experimental.pallas import tpu_sc as plsc

NC = 2
NS = 16
LANES = 16
NB = 5


def _dist_body(n_nodes, n_edges, chunk,
               rx_hbm, ry_hbm, rz_hbm, ox_hbm, oy_hbm, oz_hbm,
               ii_hbm, jj_hbm, out_hbm,
               rx_s, ry_s, rz_s,
               ii_v, jj_v, ox_v, oy_v, oz_v,
               gxi, gyi, gzi, gxj, gyj, gzj, out_v,
               sem_load, sem_gath, sem_out):
    cid = lax.axis_index("c")
    sid = lax.axis_index("s")
    wid = cid * NS + sid

    @pl.when(sid == 0)
    def _stage():
        pltpu.sync_copy(rx_hbm, rx_s)
        pltpu.sync_copy(ry_hbm, ry_s)
        pltpu.sync_copy(rz_hbm, rz_s)

    plsc.subcore_barrier()

    ept = n_edges // (NC * NS)
    nch = ept // chunk
    groups = chunk // LANES
    assert chunk % LANES == 0 and ept % chunk == 0 and nch % NB == 0
    tile0 = wid * ept

    lin_pairs = lambda b: (
        (ii_hbm, ii_v[b]), (jj_hbm, jj_v[b]),
        (ox_hbm, ox_v[b]), (oy_hbm, oy_v[b]), (oz_hbm, oz_v[b]))

    def fire_loads(ci, b):
        base = tile0 + ci * chunk
        for src, dst in lin_pairs(b):
            pltpu.async_copy(src.at[pl.ds(base, chunk)], dst, sem_load.at[b])

    def wait_loads(b):
        for src, dst in lin_pairs(b):
            pltpu.make_async_copy(src.at[pl.ds(0, chunk)], dst, sem_load.at[b]).wait()

    def gath_triples(b):
        return ((rx_s, ii_v[b], gxi[b]), (ry_s, ii_v[b], gyi[b]),
                (rz_s, ii_v[b], gzi[b]), (rx_s, jj_v[b], gxj[b]),
                (ry_s, jj_v[b], gyj[b]), (rz_s, jj_v[b], gzj[b]))

    def fire_gathers(b):
        for tab, idx, dst in gath_triples(b):
            pltpu.async_copy(tab.at[idx], dst, sem_gath.at[b])

    def wait_gathers(b):
        for tab, idx, dst in gath_triples(b):
            pltpu.make_async_copy(tab.at[idx], dst, sem_gath.at[b]).wait()

    def fire_store(ci, b):
        base = tile0 + ci * chunk
        pltpu.async_copy(out_v[b], out_hbm.at[pl.ds(base, chunk)], sem_out.at[b])

    def wait_store(b):
        pltpu.make_async_copy(out_v[b], out_hbm.at[pl.ds(0, chunk)], sem_out.at[b]).wait()

    def compute(b):
        def group_body(g, _):
            lin = g * LANES
            xi = gxi[b][pl.ds(lin, LANES)]
            yi = gyi[b][pl.ds(lin, LANES)]
            zi = gzi[b][pl.ds(lin, LANES)]
            xj = gxj[b][pl.ds(lin, LANES)]
            yj = gyj[b][pl.ds(lin, LANES)]
            zj = gzj[b][pl.ds(lin, LANES)]
            ox = ox_v[b][pl.ds(lin, LANES)]
            oy = oy_v[b][pl.ds(lin, LANES)]
            oz = oz_v[b][pl.ds(lin, LANES)]
            dx = xi - (xj + ox)
            dy = yi - (yj + oy)
            dz = zi - (zj + oz)
            s = dx * dx + dy * dy + dz * dz
            s = jnp.maximum(s, jnp.float32(1e-30))
            i = plsc.bitcast(s, jnp.int32)
            i = jnp.int32(0x5F3759DF) - (i >> 1)
            y = plsc.bitcast(i, jnp.float32)
            hs = s * jnp.float32(0.5)
            y = y * (jnp.float32(1.5) - hs * y * y)
            y = y * (jnp.float32(1.5) - hs * y * y)
            y = y * (jnp.float32(1.5) - hs * y * y)
            out_v[b][pl.ds(lin, LANES)] = s * y
            return ()

        lax.fori_loop(0, groups, group_body, (), unroll=4)

    for b in range(NB - 1):
        fire_loads(b, b)
    wait_loads(0)
    fire_gathers(0)

    def step(s, _):
        for b in range(NB):
            ci = s * NB + b

            wait_gathers(b)

            bn = (b + 1) % NB

            @pl.when(ci + 1 < nch)
            def _next_gath():
                wait_loads(bn)
                fire_gathers(bn)

            bl = (b + NB - 1) % NB

            @pl.when(ci + (NB - 1) < nch)
            def _next_loads():
                fire_loads(ci + (NB - 1), bl)

            @pl.when(ci >= NB)
            def _drain_store():
                wait_store(b)

            compute(b)
            fire_store(ci, b)
        return ()

    lax.fori_loop(0, nch // NB, step, ())

    for b in range(NB):
        wait_store(b)


@functools.partial(jax.jit, static_argnames=("n_nodes", "n_edges", "chunk"))
def _dist(rx, ry, rz, ox, oy, oz, ii, jj, *, n_nodes, n_edges, chunk):
    mesh = plsc.VectorSubcoreMesh(
        core_axis_name="c", subcore_axis_name="s",
        num_cores=NC, num_subcores=NS)
    body = functools.partial(_dist_body, n_nodes, n_edges, chunk)
    vf = lambda: [pltpu.VMEM((chunk,), jnp.float32) for _ in range(NB)]
    vi = lambda: [pltpu.VMEM((chunk,), jnp.int32) for _ in range(NB)]
    return pl.kernel(
        body,
        out_type=jax.ShapeDtypeStruct((n_edges,), jnp.float32),
        mesh=mesh,
        compiler_params=pltpu.CompilerParams(needs_layout_passes=False),
        scratch_types=[
            pltpu.VMEM_SHARED((n_nodes,), jnp.float32),
            pltpu.VMEM_SHARED((n_nodes,), jnp.float32),
            pltpu.VMEM_SHARED((n_nodes,), jnp.float32),
            vi(), vi(), vf(), vf(), vf(),
            vf(), vf(), vf(), vf(), vf(), vf(), vf(),
            pltpu.SemaphoreType.DMA((NB,)),
            pltpu.SemaphoreType.DMA((NB,)),
            pltpu.SemaphoreType.DMA((NB,)),
        ],
    )(rx, ry, rz, ox, oy, oz, ii, jj)


def kernel(r, offsets, idx_i, idx_j):
    r = r.astype(jnp.float32)
    offsets = offsets.astype(jnp.float32)
    n_nodes = r.shape[0]
    n_edges = idx_i.shape[0]
    rx, ry, rz = r[:, 0], r[:, 1], r[:, 2]
    ox, oy, oz = offsets[:, 0], offsets[:, 1], offsets[:, 2]
    ii = idx_i.astype(jnp.int32)
    jj = idx_j.astype(jnp.int32)
    out = _dist(rx, ry, rz, ox, oy, oz, ii, jj,
                n_nodes=n_nodes, n_edges=n_edges, chunk=1600)
    return out.reshape(n_edges, 1)
